# Initial kernel scaffold; baseline (speedup 1.0000x reference)
import jax
import jax.numpy as jnp
from jax.experimental import pallas as pl


def kernel(xyz, center_idx):
    raise NotImplementedError("write your pallas kernel here")


# speedup vs baseline: 3.6910x; 3.6910x over previous
import functools

import jax
import jax.numpy as jnp
from jax import lax
from jax.experimental import pallas as pl

B, N, C = 8, 8192, 3
NUM_GROUP = 512
GROUP_SIZE = 32
GB = 128
NGB = NUM_GROUP // GB


def _group_kernel(idx_ref, xyzt_ref, xyzp_ref, neigh_ref, cent_ref):
    f32 = jnp.float32
    idx_row = idx_ref[0, 0, :]
    xyzt = xyzt_ref[0]
    xyzp = xyzp_ref[0]

    iota_n = lax.broadcasted_iota(jnp.int32, (GB, N), 1)
    xr = xyzt[0:1, :]
    yr = xyzt[1:2, :]
    zr = xyzt[2:3, :]

    onehot_c = idx_row[:, None] == iota_n
    zero = jnp.float32(0.0)
    cx = jnp.sum(jnp.where(onehot_c, xr, zero), axis=1, keepdims=True)
    cy = jnp.sum(jnp.where(onehot_c, yr, zero), axis=1, keepdims=True)
    cz = jnp.sum(jnp.where(onehot_c, zr, zero), axis=1, keepdims=True)
    centers = jnp.concatenate([cx, cy, cz], axis=1)

    xn2 = xr * xr + yr * yr + zr * zr
    cn2 = cx * cx + cy * cy + cz * cz
    e = jnp.dot(centers, xyzt, preferred_element_type=f32)
    dist = (-2.0 * e + cn2) + xn2

    sels = []
    big = jnp.int32(N)
    inf = jnp.float32(jnp.inf)
    for _ in range(GROUP_SIZE):
        m = jnp.min(dist, axis=1, keepdims=True)
        am = jnp.where(dist == m, iota_n, big)
        j = jnp.min(am, axis=1, keepdims=True)
        onehot = iota_n == j
        sx = jnp.sum(jnp.where(onehot, xr, zero), axis=1, keepdims=True)
        sy = jnp.sum(jnp.where(onehot, yr, zero), axis=1, keepdims=True)
        sz = jnp.sum(jnp.where(onehot, zr, zero), axis=1, keepdims=True)
        dist = jnp.where(onehot, inf, dist)
        sels.append(jnp.concatenate([sx - cx, sy - cy, sz - cz], axis=1))
    neigh_ref[0] = jnp.concatenate(sels, axis=1)
    cent_ref[0] = centers


def kernel(xyz, center_idx):
    xyzt = jnp.transpose(xyz, (0, 2, 1))
    idx3 = center_idx.reshape(B * NGB, 1, GB)

    grid = (B, NGB)
    neigh, cent = pl.pallas_call(
        _group_kernel,
        grid=grid,
        in_specs=[
            pl.BlockSpec((1, 1, GB), lambda b, g: (b * NGB + g, 0, 0)),
            pl.BlockSpec((1, 3, N), lambda b, g: (b, 0, 0)),
            pl.BlockSpec((1, N, 3), lambda b, g: (b, 0, 0)),
        ],
        out_specs=[
            pl.BlockSpec((1, GB, GROUP_SIZE * 3), lambda b, g: (b * NGB + g, 0, 0)),
            pl.BlockSpec((1, GB, 3), lambda b, g: (b * NGB + g, 0, 0)),
        ],
        out_shape=[
            jax.ShapeDtypeStruct((B * NGB, GB, GROUP_SIZE * 3), jnp.float32),
            jax.ShapeDtypeStruct((B * NGB, GB, 3), jnp.float32),
        ],
    )(idx3, xyzt, xyz)

    neighborhood = neigh.reshape(B, NUM_GROUP, GROUP_SIZE, 3)
    centers = cent.reshape(B, NUM_GROUP, 3)
    return (neighborhood, centers)


# speedup vs baseline: 10.1023x; 2.7370x over previous
import functools

import jax
import jax.numpy as jnp
from jax import lax
from jax.experimental import pallas as pl
from jax.experimental.pallas import tpu as pltpu
from jax.experimental.pallas import tpu_sc as plsc

B, N, C = 8, 8192, 3
NUM_GROUP = 512
GROUP_SIZE = 32
GB = 128
NGB = NUM_GROUP // GB
NW = 32
TOTAL_ROWS = B * NUM_GROUP * GROUP_SIZE


def _topk_kernel(idx_ref, xyzt_ref, knn_ref, cent_ref):
    f32 = jnp.float32
    idx_row = idx_ref[0, 0, :]
    xyzt = xyzt_ref[0]

    iota_n = lax.broadcasted_iota(jnp.int32, (GB, N), 1)
    iota_f = iota_n.astype(jnp.float32)
    xr = xyzt[0:1, :]
    yr = xyzt[1:2, :]
    zr = xyzt[2:3, :]

    onehot_c = idx_row[:, None] == iota_n
    zero = jnp.float32(0.0)
    cx = jnp.sum(jnp.where(onehot_c, xr, zero), axis=1, keepdims=True)
    cy = jnp.sum(jnp.where(onehot_c, yr, zero), axis=1, keepdims=True)
    cz = jnp.sum(jnp.where(onehot_c, zr, zero), axis=1, keepdims=True)
    centers = jnp.concatenate([cx, cy, cz], axis=1)

    xn2 = xr * xr + yr * yr + zr * zr
    cn2 = cx * cx + cy * cy + cz * cz
    e = jnp.dot(centers, xyzt, preferred_element_type=f32)
    dist = (-2.0 * e + cn2) + xn2

    js = []
    bigf = jnp.float32(1e9)
    inf = jnp.float32(jnp.inf)
    for _ in range(GROUP_SIZE):
        m = jnp.min(dist, axis=1, keepdims=True)
        am = jnp.where(dist == m, iota_f, bigf)
        j = jnp.min(am, axis=1, keepdims=True)
        ji = j.astype(jnp.int32)
        dist = jnp.where(iota_n == ji, inf, dist)
        js.append(ji)
    knn_ref[0] = jnp.concatenate(js, axis=1)
    cent_ref[0] = centers


CPW = NUM_GROUP // NGB
IDX_PER_W = CPW * GROUP_SIZE
OUT_PER_W = IDX_PER_W * C
XYZ_PER_B = N * C


def _make_gather():
    mesh = plsc.VectorSubcoreMesh(core_axis_name="c", subcore_axis_name="s")

    @functools.partial(
        pl.kernel, mesh=mesh,
        compiler_params=pltpu.CompilerParams(needs_layout_passes=False),
        out_type=jax.ShapeDtypeStruct((TOTAL_ROWS * C,), jnp.float32),
        scratch_types=[
            pltpu.VMEM((IDX_PER_W,), jnp.int32),
            pltpu.VMEM((XYZ_PER_B,), jnp.float32),
            pltpu.VMEM((CPW * C,), jnp.float32),
            pltpu.VMEM((OUT_PER_W,), jnp.float32),
        ],
    )
    def _gather(xyz_hbm, knn_hbm, cent_hbm, out_hbm, idx_v, xyz_v, cent_v, out_v):
        wid = lax.axis_index("s") * 2 + lax.axis_index("c")
        b = wid // NGB
        pltpu.sync_copy(knn_hbm.at[pl.ds(wid * IDX_PER_W, IDX_PER_W)], idx_v)
        pltpu.sync_copy(xyz_hbm.at[pl.ds(b * XYZ_PER_B, XYZ_PER_B)], xyz_v)
        pltpu.sync_copy(cent_hbm.at[pl.ds(wid * CPW * C, CPW * C)], cent_v)

        iota16 = lax.broadcasted_iota(jnp.int32, (16,), 0)
        three = jnp.int32(3)
        rowpats = [lax.div(iota16 + 16 * u, three) for u in range(6)]
        coordpats = [lax.rem(iota16 + 16 * u, three) for u in range(6)]

        def body(c, _):
            cbase32 = jnp.broadcast_to(c * GROUP_SIZE, (16,))
            cbase3 = jnp.broadcast_to(c * C, (16,))
            obase = c * (GROUP_SIZE * C)
            for u in range(6):
                rowsel = plsc.load_gather(idx_v, [cbase32 + rowpats[u]])
                el = rowsel * three + coordpats[u]
                val = plsc.load_gather(xyz_v, [el])
                cvec = plsc.load_gather(cent_v, [cbase3 + coordpats[u]])
                out_v[pl.ds(obase + 16 * u, 16)] = val - cvec
            return _

        lax.fori_loop(0, CPW, body, None)
        pltpu.sync_copy(out_v, out_hbm.at[pl.ds(wid * OUT_PER_W, OUT_PER_W)])

    return _gather


_gather_rows = _make_gather()


def kernel(xyz, center_idx):
    xyzt = jnp.transpose(xyz, (0, 2, 1))
    idx3 = center_idx.reshape(B * NGB, 1, GB)

    knn, cent = pl.pallas_call(
        _topk_kernel,
        grid=(B, NGB),
        in_specs=[
            pl.BlockSpec((1, 1, GB), lambda b, g: (b * NGB + g, 0, 0)),
            pl.BlockSpec((1, 3, N), lambda b, g: (b, 0, 0)),
        ],
        out_specs=[
            pl.BlockSpec((1, GB, GROUP_SIZE), lambda b, g: (b * NGB + g, 0, 0)),
            pl.BlockSpec((1, GB, 3), lambda b, g: (b * NGB + g, 0, 0)),
        ],
        out_shape=[
            jax.ShapeDtypeStruct((B * NGB, GB, GROUP_SIZE), jnp.int32),
            jax.ShapeDtypeStruct((B * NGB, GB, 3), jnp.float32),
        ],
    )(idx3, xyzt)

    neigh = _gather_rows(
        xyz.reshape(B * N * C),
        knn.reshape(TOTAL_ROWS),
        cent.reshape(B * NUM_GROUP * C),
    )

    neighborhood = neigh.reshape(B, NUM_GROUP, GROUP_SIZE, C)
    centers = cent.reshape(B, NUM_GROUP, C)
    return (neighborhood, centers)
